# baseline (speedup 1.0000x reference)
import functools

import jax
import jax.numpy as jnp
from jax import lax
from jax.experimental import pallas as pl
from jax.experimental.pallas import tpu as pltpu
from jax.experimental.pallas import tpu_sc as plsc

NC = 2
NS = 16
NW = NC * NS


def _proj1(x, W1_l, W1_r, b1, n_pad):
    n, _ = x.shape
    d_h = W1_l.shape[1]

    def body(x_ref, wl_ref, wr_ref, b_ref, y_ref, z_ref):
        xx = x_ref[...]
        zeros = jnp.zeros((n_pad - n, d_h), jnp.float32)
        y_ref[0:n, :] = jnp.dot(
            xx, wl_ref[...], preferred_element_type=jnp.float32
        )
        y_ref[n:n_pad, :] = zeros
        z_ref[0:n, :] = (
            jnp.dot(xx, wr_ref[...], preferred_element_type=jnp.float32)
            + b_ref[...]
        )
        z_ref[n:n_pad, :] = zeros

    return pl.pallas_call(
        body,
        out_shape=[
            jax.ShapeDtypeStruct((n_pad, d_h), jnp.float32),
            jax.ShapeDtypeStruct((n_pad, d_h), jnp.float32),
        ],
    )(x, W1_l, W1_r, b1.reshape(1, d_h))


def _combine1(s1p, degp, z1):
    n, d_h = z1.shape

    def body(s_ref, d_ref, z_ref, h_ref, r_ref):
        s = s_ref[0] + s_ref[1]
        r = 1.0 / jnp.maximum(d_ref[0] + d_ref[1], 1.0)
        h_ref[...] = jnp.maximum(s * r + z_ref[...], 0.0)
        r_ref[...] = r

    return pl.pallas_call(
        body,
        out_shape=[
            jax.ShapeDtypeStruct((n, d_h), jnp.float32),
            jax.ShapeDtypeStruct((n, d_h), jnp.float32),
        ],
    )(s1p, degp, z1)


def _combine2(s2p, rdeg, h, W2_l, W2_r, b2, n):
    d_out = W2_l.shape[1]

    def body(s_ref, r_ref, h_ref, wl_ref, wr_ref, b_ref, o_ref):
        m = (s_ref[0, 0:n] + s_ref[1, 0:n]) * r_ref[0:n, :]
        o_ref[...] = (
            jnp.dot(m, wl_ref[...], preferred_element_type=jnp.float32)
            + jnp.dot(
                h_ref[0:n, :], wr_ref[...], preferred_element_type=jnp.float32
            )
            + b_ref[...]
        )

    return pl.pallas_call(
        body,
        out_shape=jax.ShapeDtypeStruct((n, d_out), jnp.float32),
    )(s2p, rdeg, h, W2_l, W2_r, b2.reshape(1, d_out))


@functools.partial(jax.jit, static_argnames=("with_deg",))
def _sc_segment_sum(y, srcr, dstr, *, with_deg):
    n, d = y.shape
    _, num_chunks, batch = srcr.shape
    rows_per_tile = n // NS

    mesh = plsc.VectorSubcoreMesh(
        core_axis_name="c", subcore_axis_name="s", num_cores=NC, num_subcores=NS
    )
    assert num_chunks % 2 == 0
    out_type = [jax.ShapeDtypeStruct((NC, n, d), jnp.float32)]
    scratch = [
        pltpu.VMEM((num_chunks, batch), jnp.int32),
        pltpu.VMEM((num_chunks, batch), jnp.int32),
        pltpu.VMEM((batch, d), jnp.float32),
        pltpu.VMEM((batch, d), jnp.float32),
        pltpu.VMEM((rows_per_tile, d), jnp.float32),
        pltpu.VMEM_SHARED((n, d), jnp.float32),
        pltpu.SemaphoreType.DMA,
        pltpu.SemaphoreType.DMA,
        pltpu.SemaphoreType.DMA,
        pltpu.SemaphoreType.DMA,
    ]
    if with_deg:
        out_type.append(jax.ShapeDtypeStruct((NC, n, d), jnp.float32))
        scratch.append(pltpu.VMEM((batch, d), jnp.float32))
        scratch.append(pltpu.VMEM_SHARED((n, d), jnp.float32))

    @functools.partial(
        pl.kernel, out_type=out_type, mesh=mesh, scratch_types=scratch,
        compiler_params=pltpu.CompilerParams(use_tc_tiling_on_sc=False),
    )
    def k(y_hbm, src_hbm, dst_hbm, *refs):
        if with_deg:
            (s_out, deg_out, idxs_v, idxd_v, rows_a, rows_b, zbuf_v, acc_s,
             sem_a, sem_b, ssem_a, ssem_b, ones_v, acc_d) = refs
        else:
            (s_out, idxs_v, idxd_v, rows_a, rows_b, zbuf_v, acc_s,
             sem_a, sem_b, ssem_a, ssem_b) = refs

        cid = lax.axis_index("c")
        sid = lax.axis_index("s")
        wid = cid * NS + sid
        base = sid * rows_per_tile

        zv = jnp.zeros((d,), jnp.float32)

        @pl.loop(0, rows_per_tile)
        def _(i):
            zbuf_v[i, :] = zv

        if with_deg:
            ov = jnp.ones((d,), jnp.float32)

            @pl.loop(0, batch)
            def _(i):
                ones_v[i, :] = ov

        pltpu.sync_copy(zbuf_v, acc_s.at[pl.ds(base, rows_per_tile)])
        if with_deg:
            pltpu.sync_copy(zbuf_v, acc_d.at[pl.ds(base, rows_per_tile)])
        pltpu.sync_copy(src_hbm.at[wid], idxs_v)
        pltpu.sync_copy(dst_hbm.at[wid], idxd_v)
        plsc.subcore_barrier()

        pltpu.async_copy(y_hbm.at[idxs_v.at[0]], rows_a, sem_a)

        @pl.loop(0, num_chunks, step=2)
        def _(kk):
            @pl.when(kk > 0)
            def _():
                pltpu.make_async_copy(
                    rows_b, acc_s.at[idxd_v.at[kk]], ssem_b
                ).wait()

            pltpu.async_copy(y_hbm.at[idxs_v.at[kk + 1]], rows_b, sem_b)
            pltpu.make_async_copy(y_hbm.at[idxs_v.at[kk]], rows_a, sem_a).wait()
            pltpu.async_copy(rows_a, acc_s.at[idxd_v.at[kk]], ssem_a, add=True)
            if with_deg:
                pltpu.sync_copy(ones_v, acc_d.at[idxd_v.at[kk]], add=True)

            pltpu.make_async_copy(
                y_hbm.at[idxs_v.at[kk + 1]], rows_b, sem_b
            ).wait()
            pltpu.async_copy(
                rows_b, acc_s.at[idxd_v.at[kk + 1]], ssem_b, add=True
            )
            if with_deg:
                pltpu.sync_copy(ones_v, acc_d.at[idxd_v.at[kk + 1]], add=True)

            pltpu.make_async_copy(rows_a, acc_s.at[idxd_v.at[kk]], ssem_a).wait()

            @pl.when(kk + 2 < num_chunks)
            def _():
                pltpu.async_copy(y_hbm.at[idxs_v.at[kk + 2]], rows_a, sem_a)

        pltpu.make_async_copy(
            rows_b, acc_s.at[idxd_v.at[num_chunks - 1]], ssem_b
        ).wait()

        plsc.subcore_barrier()
        sl = pl.ds(base, rows_per_tile)
        pltpu.sync_copy(acc_s.at[sl], s_out.at[cid, sl])
        if with_deg:
            pltpu.sync_copy(acc_d.at[sl], deg_out.at[cid, sl])

    return k(y, srcr, dstr)


def kernel(x, edge_index, W1_l, b1, W1_r, W2_l, b2, W2_r):
    n = x.shape[0]
    e = edge_index.shape[1]
    per_tile = e // NW
    assert per_tile * NW == e
    batch = 128
    while per_tile % batch:
        batch -= 1
    num_chunks = per_tile // batch
    n_pad = -(-n // (NS * 8)) * (NS * 8)

    srcr = edge_index[0].reshape(NW, num_chunks, batch)
    dstr = edge_index[1].reshape(NW, num_chunks, batch)

    y1, z1 = _proj1(x, W1_l, W1_r, b1, n_pad)
    s1p, degp = _sc_segment_sum(y1, srcr, dstr, with_deg=True)
    h, rdeg = _combine1(s1p, degp, z1)
    (s2p,) = _sc_segment_sum(h, srcr, dstr, with_deg=False)
    return _combine2(s2p, rdeg, h, W2_l, W2_r, b2, n)


# speedup vs baseline: 1.2049x; 1.2049x over previous
import functools

import jax
import jax.numpy as jnp
from jax import lax
from jax.experimental import pallas as pl
from jax.experimental.pallas import tpu as pltpu
from jax.experimental.pallas import tpu_sc as plsc

NC = 2
NS = 16
NW = NC * NS


def _proj1(x, W1_l, W1_r, b1, n_pad):
    n, _ = x.shape
    d_h = W1_l.shape[1]

    def body(x_ref, wl_ref, wr_ref, b_ref, y_ref, z_ref):
        xx = x_ref[...]
        zeros = jnp.zeros((n_pad - n, d_h), jnp.float32)
        y_ref[0:n, :] = jnp.dot(
            xx, wl_ref[...], preferred_element_type=jnp.float32
        )
        y_ref[n:n_pad, :] = zeros
        z_ref[0:n, :] = (
            jnp.dot(xx, wr_ref[...], preferred_element_type=jnp.float32)
            + b_ref[...]
        )
        z_ref[n:n_pad, :] = zeros

    return pl.pallas_call(
        body,
        out_shape=[
            jax.ShapeDtypeStruct((n_pad, d_h), jnp.float32),
            jax.ShapeDtypeStruct((n_pad, d_h), jnp.float32),
        ],
    )(x, W1_l, W1_r, b1.reshape(1, d_h))


def _combine1(s1p, degp, z1):
    n, d_h = z1.shape

    def body(s_ref, d_ref, z_ref, h_ref, r_ref):
        s = s_ref[0] + s_ref[1]
        r = 1.0 / jnp.maximum(d_ref[0] + d_ref[1], 1.0)
        h_ref[...] = jnp.maximum(s * r + z_ref[...], 0.0)
        r_ref[...] = r

    return pl.pallas_call(
        body,
        out_shape=[
            jax.ShapeDtypeStruct((n, d_h), jnp.float32),
            jax.ShapeDtypeStruct((n, d_h), jnp.float32),
        ],
    )(s1p, degp, z1)


def _combine2(s2p, rdeg, h, W2_l, W2_r, b2, n):
    d_out = W2_l.shape[1]

    def body(s_ref, r_ref, h_ref, wl_ref, wr_ref, b_ref, o_ref):
        m = (s_ref[0, 0:n] + s_ref[1, 0:n]) * r_ref[0:n, :]
        o_ref[...] = (
            jnp.dot(m, wl_ref[...], preferred_element_type=jnp.float32)
            + jnp.dot(
                h_ref[0:n, :], wr_ref[...], preferred_element_type=jnp.float32
            )
            + b_ref[...]
        )

    return pl.pallas_call(
        body,
        out_shape=jax.ShapeDtypeStruct((n, d_out), jnp.float32),
    )(s2p, rdeg, h, W2_l, W2_r, b2.reshape(1, d_out))


@functools.partial(jax.jit, static_argnames=("with_deg",))
def _sc_segment_sum(y, srcr, dstr, *, with_deg):
    n, d = y.shape
    _, num_chunks, batch = srcr.shape
    rows_per_tile = n // NS

    mesh = plsc.VectorSubcoreMesh(
        core_axis_name="c", subcore_axis_name="s", num_cores=NC, num_subcores=NS
    )
    assert num_chunks % 2 == 0
    out_type = [jax.ShapeDtypeStruct((NC, n, d), jnp.float32)]
    scratch = [
        pltpu.VMEM((num_chunks, batch), jnp.int32),
        pltpu.VMEM((num_chunks, batch), jnp.int32),
        pltpu.VMEM((batch, d), jnp.float32),
        pltpu.VMEM((batch, d), jnp.float32),
        pltpu.VMEM((rows_per_tile, d), jnp.float32),
        pltpu.VMEM_SHARED((n, d), jnp.float32),
        pltpu.SemaphoreType.DMA,
        pltpu.SemaphoreType.DMA,
        pltpu.SemaphoreType.DMA,
        pltpu.SemaphoreType.DMA,
    ]
    if with_deg:
        out_type.append(jax.ShapeDtypeStruct((NC, n, d), jnp.float32))
        scratch.append(pltpu.VMEM((batch, d), jnp.float32))
        scratch.append(pltpu.VMEM_SHARED((n, d), jnp.float32))

    @functools.partial(
        pl.kernel, out_type=out_type, mesh=mesh, scratch_types=scratch,
        compiler_params=pltpu.CompilerParams(use_tc_tiling_on_sc=False),
    )
    def k(y_hbm, src_hbm, dst_hbm, *refs):
        if with_deg:
            (s_out, deg_out, idxs_v, idxd_v, rows_a, rows_b, zbuf_v, acc_s,
             sem_a, sem_b, ssem_a, ssem_b, ones_v, acc_d) = refs
        else:
            (s_out, idxs_v, idxd_v, rows_a, rows_b, zbuf_v, acc_s,
             sem_a, sem_b, ssem_a, ssem_b) = refs

        cid = lax.axis_index("c")
        sid = lax.axis_index("s")
        wid = cid * NS + sid
        base = sid * rows_per_tile

        zv = jnp.zeros((d,), jnp.float32)

        @pl.loop(0, rows_per_tile)
        def _(i):
            zbuf_v[i, :] = zv

        if with_deg:
            ov = jnp.ones((d,), jnp.float32)

            @pl.loop(0, batch)
            def _(i):
                ones_v[i, :] = ov

        pltpu.sync_copy(zbuf_v, acc_s.at[pl.ds(base, rows_per_tile)])
        if with_deg:
            pltpu.sync_copy(zbuf_v, acc_d.at[pl.ds(base, rows_per_tile)])
        pltpu.sync_copy(src_hbm.at[wid], idxs_v)
        pltpu.sync_copy(dst_hbm.at[wid], idxd_v)
        plsc.subcore_barrier()

        pltpu.async_copy(y_hbm.at[idxs_v.at[0]], rows_a, sem_a)

        @pl.loop(0, num_chunks, step=2)
        def _(kk):
            @pl.when(kk > 0)
            def _():
                pltpu.make_async_copy(
                    rows_b, acc_s.at[idxd_v.at[kk]], ssem_b
                ).wait()

            pltpu.async_copy(y_hbm.at[idxs_v.at[kk + 1]], rows_b, sem_b)
            pltpu.make_async_copy(y_hbm.at[idxs_v.at[kk]], rows_a, sem_a).wait()
            pltpu.async_copy(rows_a, acc_s.at[idxd_v.at[kk]], ssem_a, add=True)
            if with_deg:
                pltpu.sync_copy(ones_v, acc_d.at[idxd_v.at[kk]], add=True)

            pltpu.make_async_copy(
                y_hbm.at[idxs_v.at[kk + 1]], rows_b, sem_b
            ).wait()
            pltpu.async_copy(
                rows_b, acc_s.at[idxd_v.at[kk + 1]], ssem_b, add=True
            )
            if with_deg:
                pltpu.sync_copy(ones_v, acc_d.at[idxd_v.at[kk + 1]], add=True)

            pltpu.make_async_copy(rows_a, acc_s.at[idxd_v.at[kk]], ssem_a).wait()

            @pl.when(kk + 2 < num_chunks)
            def _():
                pltpu.async_copy(y_hbm.at[idxs_v.at[kk + 2]], rows_a, sem_a)

        pltpu.make_async_copy(
            rows_b, acc_s.at[idxd_v.at[num_chunks - 1]], ssem_b
        ).wait()

        plsc.subcore_barrier()
        sl = pl.ds(base, rows_per_tile)
        pltpu.sync_copy(acc_s.at[sl], s_out.at[cid, sl])
        if with_deg:
            pltpu.sync_copy(acc_d.at[sl], deg_out.at[cid, sl])

    return k(y, srcr, dstr)


def kernel(x, edge_index, W1_l, b1, W1_r, W2_l, b2, W2_r):
    n = x.shape[0]
    e = edge_index.shape[1]
    per_tile = e // NW
    assert per_tile * NW == e
    batch = max(
        b for b in range(1, 1251)
        if per_tile % b == 0 and (per_tile // b) % 2 == 0
    )
    num_chunks = per_tile // batch
    n_pad = -(-n // (NS * 8)) * (NS * 8)

    srcr = edge_index[0].reshape(NW, num_chunks, batch)
    dstr = edge_index[1].reshape(NW, num_chunks, batch)

    y1, z1 = _proj1(x, W1_l, W1_r, b1, n_pad)
    s1p, degp = _sc_segment_sum(y1, srcr, dstr, with_deg=True)
    h, rdeg = _combine1(s1p, degp, z1)
    (s2p,) = _sc_segment_sum(h, srcr, dstr, with_deg=False)
    return _combine2(s2p, rdeg, h, W2_l, W2_r, b2, n)


# speedup vs baseline: 1.3523x; 1.1223x over previous
import functools

import jax
import jax.numpy as jnp
from jax import lax
from jax.experimental import pallas as pl
from jax.experimental.pallas import tpu as pltpu
from jax.experimental.pallas import tpu_sc as plsc

NC = 2
NS = 16
NW = NC * NS


def _proj1(x, W1_l, W1_r, b1, n_pad):
    n, _ = x.shape
    d_h = W1_l.shape[1]

    def body(x_ref, wl_ref, wr_ref, b_ref, y_ref, z_ref):
        xx = x_ref[...]
        zeros = jnp.zeros((n_pad - n, d_h), jnp.float32)
        y_ref[0:n, :] = jnp.dot(
            xx, wl_ref[...], preferred_element_type=jnp.float32
        )
        y_ref[n:n_pad, :] = zeros
        z_ref[0:n, :] = (
            jnp.dot(xx, wr_ref[...], preferred_element_type=jnp.float32)
            + b_ref[...]
        )
        z_ref[n:n_pad, :] = zeros

    return pl.pallas_call(
        body,
        out_shape=[
            jax.ShapeDtypeStruct((n_pad, d_h), jnp.float32),
            jax.ShapeDtypeStruct((n_pad, d_h), jnp.float32),
        ],
    )(x, W1_l, W1_r, b1.reshape(1, d_h))


def _combine1(s1p, degp, z1):
    n, d_h = z1.shape

    def body(s_ref, d_ref, z_ref, h_ref, r_ref):
        s = s_ref[0] + s_ref[1]
        r = 1.0 / jnp.maximum(d_ref[0] + d_ref[1], 1.0)
        h_ref[...] = jnp.maximum(s * r + z_ref[...], 0.0)
        r_ref[...] = r

    return pl.pallas_call(
        body,
        out_shape=[
            jax.ShapeDtypeStruct((n, d_h), jnp.float32),
            jax.ShapeDtypeStruct((n, d_h), jnp.float32),
        ],
    )(s1p, degp, z1)


def _combine2(s2p, rdeg, h, W2_l, W2_r, b2, n):
    d_out = W2_l.shape[1]

    def body(s_ref, r_ref, h_ref, wl_ref, wr_ref, b_ref, o_ref):
        m = (s_ref[0, 0:n] + s_ref[1, 0:n]) * r_ref[0:n, :]
        o_ref[...] = (
            jnp.dot(m, wl_ref[...], preferred_element_type=jnp.float32)
            + jnp.dot(
                h_ref[0:n, :], wr_ref[...], preferred_element_type=jnp.float32
            )
            + b_ref[...]
        )

    return pl.pallas_call(
        body,
        out_shape=jax.ShapeDtypeStruct((n, d_out), jnp.float32),
    )(s2p, rdeg, h, W2_l, W2_r, b2.reshape(1, d_out))


@functools.partial(jax.jit, static_argnames=("batch", "with_deg"))
def _sc_segment_sum(y, edge, *, batch, with_deg):
    n, d = y.shape
    e = edge.shape[1]
    per_tile = e // NW
    num_chunks = per_tile // batch
    rows_per_tile = n // NS
    assert num_chunks % 2 == 0 and batch % 8 == 0 and per_tile % batch == 0

    mesh = plsc.VectorSubcoreMesh(
        core_axis_name="c", subcore_axis_name="s", num_cores=NC, num_subcores=NS
    )
    out_type = [jax.ShapeDtypeStruct((NC, n, d), jnp.float32)]
    scratch = [
        pltpu.VMEM((per_tile,), jnp.int32),
        pltpu.VMEM((batch,), jnp.int32),
        pltpu.VMEM((batch,), jnp.int32),
        pltpu.VMEM((batch, d), jnp.float32),
        pltpu.VMEM((batch, d), jnp.float32),
        pltpu.VMEM((rows_per_tile, d), jnp.float32),
        pltpu.VMEM_SHARED((n, d), jnp.float32),
        pltpu.SemaphoreType.DMA,
        pltpu.SemaphoreType.DMA,
        pltpu.SemaphoreType.DMA,
        pltpu.SemaphoreType.DMA,
    ]
    if with_deg:
        out_type.append(jax.ShapeDtypeStruct((NC, n, d), jnp.float32))
        scratch.append(pltpu.VMEM((batch, d), jnp.float32))
        scratch.append(pltpu.VMEM_SHARED((n, d), jnp.float32))

    @functools.partial(
        pl.kernel, out_type=out_type, mesh=mesh, scratch_types=scratch,
        compiler_params=pltpu.CompilerParams(use_tc_tiling_on_sc=False),
    )
    def k(y_hbm, edge_hbm, *refs):
        if with_deg:
            (s_out, deg_out, idxs_v, idxd_a, idxd_b, rows_a, rows_b, zbuf_v,
             acc_s, sem_a, sem_b, ssem_a, ssem_b, ones_v, acc_d) = refs
        else:
            (s_out, idxs_v, idxd_a, idxd_b, rows_a, rows_b, zbuf_v,
             acc_s, sem_a, sem_b, ssem_a, ssem_b) = refs

        cid = lax.axis_index("c")
        sid = lax.axis_index("s")
        wid = cid * NS + sid
        ebase = wid * per_tile
        base = sid * rows_per_tile

        zv = jnp.zeros((d,), jnp.float32)

        @pl.loop(0, rows_per_tile)
        def _(i):
            zbuf_v[i, :] = zv

        if with_deg:
            ov = jnp.ones((d,), jnp.float32)

            @pl.loop(0, batch)
            def _(i):
                ones_v[i, :] = ov

        pltpu.sync_copy(zbuf_v, acc_s.at[pl.ds(base, rows_per_tile)])
        if with_deg:
            pltpu.sync_copy(zbuf_v, acc_d.at[pl.ds(base, rows_per_tile)])
        pltpu.sync_copy(edge_hbm.at[0, pl.ds(ebase, per_tile)], idxs_v)
        pltpu.sync_copy(edge_hbm.at[1, pl.ds(ebase, batch)], idxd_a)
        plsc.subcore_barrier()

        pltpu.async_copy(y_hbm.at[idxs_v.at[pl.ds(0, batch)]], rows_a, sem_a)

        @pl.loop(0, num_chunks, step=2)
        def _(kk):
            @pl.when(kk > 0)
            def _():
                pltpu.make_async_copy(rows_b, acc_s.at[idxd_b], ssem_b).wait()

            pltpu.sync_copy(
                edge_hbm.at[1, pl.ds(ebase + (kk + 1) * batch, batch)], idxd_b
            )
            pltpu.async_copy(
                y_hbm.at[idxs_v.at[pl.ds((kk + 1) * batch, batch)]],
                rows_b, sem_b,
            )
            pltpu.make_async_copy(
                y_hbm.at[idxs_v.at[pl.ds(kk * batch, batch)]], rows_a, sem_a
            ).wait()
            pltpu.async_copy(rows_a, acc_s.at[idxd_a], ssem_a, add=True)
            if with_deg:
                pltpu.sync_copy(ones_v, acc_d.at[idxd_a], add=True)

            pltpu.make_async_copy(
                y_hbm.at[idxs_v.at[pl.ds((kk + 1) * batch, batch)]],
                rows_b, sem_b,
            ).wait()
            pltpu.async_copy(rows_b, acc_s.at[idxd_b], ssem_b, add=True)
            if with_deg:
                pltpu.sync_copy(ones_v, acc_d.at[idxd_b], add=True)

            pltpu.make_async_copy(rows_a, acc_s.at[idxd_a], ssem_a).wait()

            @pl.when(kk + 2 < num_chunks)
            def _():
                pltpu.sync_copy(
                    edge_hbm.at[1, pl.ds(ebase + (kk + 2) * batch, batch)],
                    idxd_a,
                )
                pltpu.async_copy(
                    y_hbm.at[idxs_v.at[pl.ds((kk + 2) * batch, batch)]],
                    rows_a, sem_a,
                )

        pltpu.make_async_copy(rows_b, acc_s.at[idxd_b], ssem_b).wait()

        plsc.subcore_barrier()
        sl = pl.ds(base, rows_per_tile)
        pltpu.sync_copy(acc_s.at[sl], s_out.at[cid, sl])
        if with_deg:
            pltpu.sync_copy(acc_d.at[sl], deg_out.at[cid, sl])

    return k(y, edge)


def kernel(x, edge_index, W1_l, b1, W1_r, W2_l, b2, W2_r):
    n = x.shape[0]
    e = edge_index.shape[1]
    per_tile = e // NW
    assert per_tile * NW == e
    batch = max(
        b for b in range(8, 1256, 8)
        if per_tile % b == 0 and (per_tile // b) % 2 == 0
    )
    n_pad = -(-n // (NS * 8)) * (NS * 8)

    y1, z1 = _proj1(x, W1_l, W1_r, b1, n_pad)
    s1p, degp = _sc_segment_sum(y1, edge_index, batch=batch, with_deg=True)
    h, rdeg = _combine1(s1p, degp, z1)
    (s2p,) = _sc_segment_sum(h, edge_index, batch=batch, with_deg=False)
    return _combine2(s2p, rdeg, h, W2_l, W2_r, b2, n)


# speedup vs baseline: 1.8510x; 1.3687x over previous
import functools

import jax
import jax.numpy as jnp
from jax import lax
from jax.experimental import pallas as pl
from jax.experimental.pallas import tpu as pltpu
from jax.experimental.pallas import tpu_sc as plsc

NC = 2
NS = 16
NW = NC * NS


def _proj1(x, W1_l, W1_r, b1, n_pad):
    n, d_in = x.shape
    d_h = W1_l.shape[1]
    p = n_pad // 8

    def body(x_ref, wl_ref, wr_ref, b_ref, y_ref, z_ref):
        xp = jnp.concatenate(
            [x_ref[...], jnp.zeros((n_pad - n, d_in), jnp.float32)], axis=0
        ).reshape(p, 8, d_in)
        bb = b_ref[...]
        ys = []
        zs = []
        for s in range(8):
            xs = xp[:, s, :]
            ys.append(jnp.dot(xs, wl_ref[...], preferred_element_type=jnp.float32))
            zs.append(
                jnp.dot(xs, wr_ref[...], preferred_element_type=jnp.float32) + bb
            )
        y_ref[...] = jnp.concatenate(ys, axis=1)
        z_ref[...] = jnp.concatenate(zs, axis=1)

    return pl.pallas_call(
        body,
        out_shape=[
            jax.ShapeDtypeStruct((p, 8 * d_h), jnp.float32),
            jax.ShapeDtypeStruct((p, 8 * d_h), jnp.float32),
        ],
    )(x, W1_l, W1_r, b1.reshape(1, d_h))


def _combine1(s1p, degp, z1):
    p, dd = z1.shape

    def body(s_ref, d_ref, z_ref, h_ref, r_ref):
        s = s_ref[0] + s_ref[1]
        r = 1.0 / jnp.maximum(d_ref[0] + d_ref[1], 1.0)
        h_ref[...] = jnp.maximum(s * r + z_ref[...], 0.0)
        r_ref[...] = r

    return pl.pallas_call(
        body,
        out_shape=[
            jax.ShapeDtypeStruct((p, dd), jnp.float32),
            jax.ShapeDtypeStruct((p, dd), jnp.float32),
        ],
    )(s1p, degp, z1)


def _combine2(s2p, rdeg, h, W2_l, W2_r, b2, n):
    d_h, d_out = W2_l.shape
    p = rdeg.shape[0]
    n_pad = p * 8

    def body(s_ref, r_ref, h_ref, wl_ref, wr_ref, b_ref, o_ref):
        m = (s_ref[0] + s_ref[1]) * r_ref[...]
        hh = h_ref[...]
        res = []
        for s in range(8):
            sl = slice(s * d_h, (s + 1) * d_h)
            res.append(
                jnp.dot(m[:, sl], wl_ref[...], preferred_element_type=jnp.float32)
                + jnp.dot(
                    hh[:, sl], wr_ref[...], preferred_element_type=jnp.float32
                )
            )
        out = jnp.stack(res, axis=1).reshape(n_pad, d_out)
        o_ref[...] = out[0:n] + b_ref[...]

    return pl.pallas_call(
        body,
        out_shape=jax.ShapeDtypeStruct((n, d_out), jnp.float32),
    )(s2p, rdeg, h, W2_l, W2_r, b2.reshape(1, d_out))


@functools.partial(jax.jit, static_argnames=("batch", "with_deg"))
def _sc_segment_sum(y, edge, *, batch, with_deg):
    n, d = y.shape
    e = edge.shape[1]
    per_tile = e // NW
    num_chunks = per_tile // batch
    rows_per_tile = n // NS
    assert num_chunks % 2 == 0 and batch % 8 == 0 and per_tile % batch == 0

    mesh = plsc.VectorSubcoreMesh(
        core_axis_name="c", subcore_axis_name="s", num_cores=NC, num_subcores=NS
    )
    out_type = [jax.ShapeDtypeStruct((NC, n, d), jnp.float32)]
    scratch = [
        pltpu.VMEM((per_tile,), jnp.int32),
        pltpu.VMEM((batch,), jnp.int32),
        pltpu.VMEM((batch,), jnp.int32),
        pltpu.VMEM((batch, d), jnp.float32),
        pltpu.VMEM((batch, d), jnp.float32),
        pltpu.VMEM((rows_per_tile, d), jnp.float32),
        pltpu.VMEM_SHARED((n, d), jnp.float32),
        pltpu.SemaphoreType.DMA,
        pltpu.SemaphoreType.DMA,
        pltpu.SemaphoreType.DMA,
        pltpu.SemaphoreType.DMA,
    ]
    if with_deg:
        out_type.append(jax.ShapeDtypeStruct((NC, n, d), jnp.float32))
        scratch.append(pltpu.VMEM((batch, d), jnp.float32))
        scratch.append(pltpu.VMEM_SHARED((n, d), jnp.float32))

    @functools.partial(
        pl.kernel, out_type=out_type, mesh=mesh, scratch_types=scratch,
        compiler_params=pltpu.CompilerParams(use_tc_tiling_on_sc=False),
    )
    def k(y_hbm, edge_hbm, *refs):
        if with_deg:
            (s_out, deg_out, idxs_v, idxd_a, idxd_b, rows_a, rows_b, zbuf_v,
             acc_s, sem_a, sem_b, ssem_a, ssem_b, ones_v, acc_d) = refs
        else:
            (s_out, idxs_v, idxd_a, idxd_b, rows_a, rows_b, zbuf_v,
             acc_s, sem_a, sem_b, ssem_a, ssem_b) = refs

        cid = lax.axis_index("c")
        sid = lax.axis_index("s")
        wid = cid * NS + sid
        ebase = wid * per_tile
        base = sid * rows_per_tile

        zv = jnp.zeros((d,), jnp.float32)

        @pl.loop(0, rows_per_tile)
        def _(i):
            zbuf_v[i, :] = zv

        if with_deg:
            ov = jnp.ones((d,), jnp.float32)

            @pl.loop(0, batch)
            def _(i):
                ones_v[i, :] = ov

        pltpu.sync_copy(zbuf_v, acc_s.at[pl.ds(base, rows_per_tile)])
        if with_deg:
            pltpu.sync_copy(zbuf_v, acc_d.at[pl.ds(base, rows_per_tile)])
        pltpu.sync_copy(edge_hbm.at[0, pl.ds(ebase, per_tile)], idxs_v)
        pltpu.sync_copy(edge_hbm.at[1, pl.ds(ebase, batch)], idxd_a)
        plsc.subcore_barrier()

        pltpu.async_copy(y_hbm.at[idxs_v.at[pl.ds(0, batch)]], rows_a, sem_a)

        @pl.loop(0, num_chunks, step=2)
        def _(kk):
            @pl.when(kk > 0)
            def _():
                pltpu.make_async_copy(rows_b, acc_s.at[idxd_b], ssem_b).wait()

            pltpu.sync_copy(
                edge_hbm.at[1, pl.ds(ebase + (kk + 1) * batch, batch)], idxd_b
            )
            pltpu.async_copy(
                y_hbm.at[idxs_v.at[pl.ds((kk + 1) * batch, batch)]],
                rows_b, sem_b,
            )
            pltpu.make_async_copy(
                y_hbm.at[idxs_v.at[pl.ds(kk * batch, batch)]], rows_a, sem_a
            ).wait()
            pltpu.async_copy(rows_a, acc_s.at[idxd_a], ssem_a, add=True)
            if with_deg:
                pltpu.sync_copy(ones_v, acc_d.at[idxd_a], add=True)

            pltpu.make_async_copy(
                y_hbm.at[idxs_v.at[pl.ds((kk + 1) * batch, batch)]],
                rows_b, sem_b,
            ).wait()
            pltpu.async_copy(rows_b, acc_s.at[idxd_b], ssem_b, add=True)
            if with_deg:
                pltpu.sync_copy(ones_v, acc_d.at[idxd_b], add=True)

            pltpu.make_async_copy(rows_a, acc_s.at[idxd_a], ssem_a).wait()

            @pl.when(kk + 2 < num_chunks)
            def _():
                pltpu.sync_copy(
                    edge_hbm.at[1, pl.ds(ebase + (kk + 2) * batch, batch)],
                    idxd_a,
                )
                pltpu.async_copy(
                    y_hbm.at[idxs_v.at[pl.ds((kk + 2) * batch, batch)]],
                    rows_a, sem_a,
                )

        pltpu.make_async_copy(rows_b, acc_s.at[idxd_b], ssem_b).wait()

        plsc.subcore_barrier()
        sl = pl.ds(base, rows_per_tile)
        pltpu.sync_copy(acc_s.at[sl], s_out.at[cid, sl])
        if with_deg:
            pltpu.sync_copy(acc_d.at[sl], deg_out.at[cid, sl])

    return k(y, edge)


def kernel(x, edge_index, W1_l, b1, W1_r, W2_l, b2, W2_r):
    n = x.shape[0]
    e = edge_index.shape[1]
    per_tile = e // NW
    assert per_tile * NW == e
    batch = max(
        b for b in range(8, 1256, 8)
        if per_tile % b == 0 and (per_tile // b) % 2 == 0
    )
    n_pad = -(-n // (NS * 8)) * (NS * 8)

    d_h = W1_l.shape[1]
    p = n_pad // 8

    y1, z1 = _proj1(x, W1_l, W1_r, b1, n_pad)
    s1p, degp = _sc_segment_sum(
        y1.reshape(n_pad, d_h), edge_index, batch=batch, with_deg=True
    )
    h, rdeg = _combine1(
        s1p.reshape(NC, p, 8 * d_h), degp.reshape(NC, p, 8 * d_h), z1
    )
    (s2p,) = _sc_segment_sum(
        h.reshape(n_pad, d_h), edge_index, batch=batch, with_deg=False
    )
    return _combine2(
        s2p.reshape(NC, p, 8 * d_h), rdeg, h, W2_l, W2_r, b2, n
    )


# speedup vs baseline: 1.9707x; 1.0647x over previous
import functools

import jax
import jax.numpy as jnp
from jax import lax
from jax.experimental import pallas as pl
from jax.experimental.pallas import tpu as pltpu
from jax.experimental.pallas import tpu_sc as plsc

NC = 2
NS = 16
NW = NC * NS


def _proj1(x, W1_l, W1_r, b1, n_pad):
    n, d_in = x.shape
    d_h = W1_l.shape[1]
    p = n_pad // 8

    def body(x_ref, wl_ref, wr_ref, b_ref, y_ref, z_ref):
        xp = jnp.concatenate(
            [x_ref[...], jnp.zeros((n_pad - n, d_in), jnp.float32)], axis=0
        ).reshape(p, 8, d_in)
        bb = b_ref[...]
        ys = []
        zs = []
        for s in range(8):
            xs = xp[:, s, :]
            ys.append(jnp.dot(xs, wl_ref[...], preferred_element_type=jnp.float32))
            zs.append(
                jnp.dot(xs, wr_ref[...], preferred_element_type=jnp.float32) + bb
            )
        y_ref[...] = jnp.concatenate(ys, axis=1)
        z_ref[...] = jnp.concatenate(zs, axis=1)

    return pl.pallas_call(
        body,
        out_shape=[
            jax.ShapeDtypeStruct((p, 8 * d_h), jnp.float32),
            jax.ShapeDtypeStruct((p, 8 * d_h), jnp.float32),
        ],
    )(x, W1_l, W1_r, b1.reshape(1, d_h))


def _combine1(s1p, degp, z1):
    p, dd = z1.shape

    def body(s_ref, d_ref, z_ref, h_ref, r_ref):
        s = s_ref[0] + s_ref[1]
        r = 1.0 / jnp.maximum(d_ref[0] + d_ref[1], 1.0)
        h_ref[...] = jnp.maximum(s * r + z_ref[...], 0.0)
        r_ref[...] = r

    return pl.pallas_call(
        body,
        out_shape=[
            jax.ShapeDtypeStruct((p, dd), jnp.float32),
            jax.ShapeDtypeStruct((p, dd), jnp.float32),
        ],
    )(s1p, degp, z1)


def _blockdiag8(W):
    d_h, d_out = W.shape
    eye = jnp.eye(8, dtype=W.dtype)
    return (eye[:, None, :, None] * W[None, :, None, :]).reshape(
        8 * d_h, 8 * d_out
    )


def _combine2(s2p, rdeg, h, Wcat, b2, n, d_out):
    p = rdeg.shape[0]
    n_pad = p * 8

    def body(s_ref, r_ref, h_ref, w_ref, b_ref, o_ref):
        m = (s_ref[0] + s_ref[1]) * r_ref[...]
        mh = jnp.concatenate([m, h_ref[...]], axis=1)
        res = jnp.dot(mh, w_ref[...], preferred_element_type=jnp.float32)
        out = jnp.stack(
            [res[:, s * d_out:(s + 1) * d_out] for s in range(8)], axis=1
        ).reshape(n_pad, d_out)
        o_ref[...] = out[0:n] + b_ref[...]

    return pl.pallas_call(
        body,
        out_shape=jax.ShapeDtypeStruct((n, d_out), jnp.float32),
    )(s2p, rdeg, h, Wcat, b2.reshape(1, d_out))


@functools.partial(jax.jit, static_argnames=("batch", "with_deg"))
def _sc_segment_sum(y, edge, *, batch, with_deg):
    n, d = y.shape
    e = edge.shape[1]
    per_tile = e // NW
    num_chunks = per_tile // batch
    rows_per_tile = n // NS
    assert num_chunks % 2 == 0 and batch % 8 == 0 and per_tile % batch == 0

    mesh = plsc.VectorSubcoreMesh(
        core_axis_name="c", subcore_axis_name="s", num_cores=NC, num_subcores=NS
    )
    out_type = [jax.ShapeDtypeStruct((NC, n, d), jnp.float32)]
    scratch = [
        pltpu.VMEM((per_tile,), jnp.int32),
        pltpu.VMEM((batch,), jnp.int32),
        pltpu.VMEM((batch,), jnp.int32),
        pltpu.VMEM((batch, d), jnp.float32),
        pltpu.VMEM((batch, d), jnp.float32),
        pltpu.VMEM((rows_per_tile, d), jnp.float32),
        pltpu.VMEM_SHARED((n, d), jnp.float32),
        pltpu.SemaphoreType.DMA,
        pltpu.SemaphoreType.DMA,
        pltpu.SemaphoreType.DMA,
        pltpu.SemaphoreType.DMA,
    ]
    if with_deg:
        out_type.append(jax.ShapeDtypeStruct((NC, n, d), jnp.float32))
        scratch.append(pltpu.VMEM((batch, d), jnp.float32))
        scratch.append(pltpu.VMEM_SHARED((n, d), jnp.float32))
        scratch.append(pltpu.SemaphoreType.DMA)
        scratch.append(pltpu.SemaphoreType.DMA)

    @functools.partial(
        pl.kernel, out_type=out_type, mesh=mesh, scratch_types=scratch,
        compiler_params=pltpu.CompilerParams(use_tc_tiling_on_sc=False),
    )
    def k(y_hbm, edge_hbm, *refs):
        if with_deg:
            (s_out, deg_out, idxs_v, idxd_a, idxd_b, rows_a, rows_b, zbuf_v,
             acc_s, sem_a, sem_b, ssem_a, ssem_b, ones_v, acc_d,
             osem_a, osem_b) = refs
        else:
            (s_out, idxs_v, idxd_a, idxd_b, rows_a, rows_b, zbuf_v,
             acc_s, sem_a, sem_b, ssem_a, ssem_b) = refs

        cid = lax.axis_index("c")
        sid = lax.axis_index("s")
        wid = cid * NS + sid
        ebase = wid * per_tile
        base = sid * rows_per_tile

        zv = jnp.zeros((d,), jnp.float32)

        @pl.loop(0, rows_per_tile)
        def _(i):
            zbuf_v[i, :] = zv

        if with_deg:
            ov = jnp.ones((d,), jnp.float32)

            @pl.loop(0, batch)
            def _(i):
                ones_v[i, :] = ov

        pltpu.sync_copy(zbuf_v, acc_s.at[pl.ds(base, rows_per_tile)])
        if with_deg:
            pltpu.sync_copy(zbuf_v, acc_d.at[pl.ds(base, rows_per_tile)])
        pltpu.sync_copy(edge_hbm.at[0, pl.ds(ebase, per_tile)], idxs_v)
        pltpu.sync_copy(edge_hbm.at[1, pl.ds(ebase, batch)], idxd_a)
        plsc.subcore_barrier()

        pltpu.async_copy(y_hbm.at[idxs_v.at[pl.ds(0, batch)]], rows_a, sem_a)

        @pl.loop(0, num_chunks, step=2)
        def _(kk):
            @pl.when(kk > 0)
            def _():
                pltpu.make_async_copy(rows_b, acc_s.at[idxd_b], ssem_b).wait()
                if with_deg:
                    pltpu.make_async_copy(
                        ones_v, acc_d.at[idxd_b], osem_b
                    ).wait()

            pltpu.sync_copy(
                edge_hbm.at[1, pl.ds(ebase + (kk + 1) * batch, batch)], idxd_b
            )
            pltpu.async_copy(
                y_hbm.at[idxs_v.at[pl.ds((kk + 1) * batch, batch)]],
                rows_b, sem_b,
            )
            pltpu.make_async_copy(
                y_hbm.at[idxs_v.at[pl.ds(kk * batch, batch)]], rows_a, sem_a
            ).wait()
            pltpu.async_copy(rows_a, acc_s.at[idxd_a], ssem_a, add=True)
            if with_deg:
                pltpu.async_copy(ones_v, acc_d.at[idxd_a], osem_a, add=True)

            pltpu.make_async_copy(
                y_hbm.at[idxs_v.at[pl.ds((kk + 1) * batch, batch)]],
                rows_b, sem_b,
            ).wait()
            pltpu.async_copy(rows_b, acc_s.at[idxd_b], ssem_b, add=True)
            if with_deg:
                pltpu.async_copy(ones_v, acc_d.at[idxd_b], osem_b, add=True)

            pltpu.make_async_copy(rows_a, acc_s.at[idxd_a], ssem_a).wait()
            if with_deg:
                pltpu.make_async_copy(ones_v, acc_d.at[idxd_a], osem_a).wait()

            @pl.when(kk + 2 < num_chunks)
            def _():
                pltpu.sync_copy(
                    edge_hbm.at[1, pl.ds(ebase + (kk + 2) * batch, batch)],
                    idxd_a,
                )
                pltpu.async_copy(
                    y_hbm.at[idxs_v.at[pl.ds((kk + 2) * batch, batch)]],
                    rows_a, sem_a,
                )

        pltpu.make_async_copy(rows_b, acc_s.at[idxd_b], ssem_b).wait()
        if with_deg:
            pltpu.make_async_copy(ones_v, acc_d.at[idxd_b], osem_b).wait()

        plsc.subcore_barrier()
        sl = pl.ds(base, rows_per_tile)
        pltpu.sync_copy(acc_s.at[sl], s_out.at[cid, sl])
        if with_deg:
            pltpu.sync_copy(acc_d.at[sl], deg_out.at[cid, sl])

    return k(y, edge)


def kernel(x, edge_index, W1_l, b1, W1_r, W2_l, b2, W2_r):
    n = x.shape[0]
    e = edge_index.shape[1]
    per_tile = e // NW
    assert per_tile * NW == e
    batch = max(
        b for b in range(8, 1256, 8)
        if per_tile % b == 0 and (per_tile // b) % 2 == 0
    )
    n_pad = -(-n // (NS * 8)) * (NS * 8)

    d_h = W1_l.shape[1]
    p = n_pad // 8

    y1, z1 = _proj1(x, W1_l, W1_r, b1, n_pad)
    s1p, degp = _sc_segment_sum(
        y1.reshape(n_pad, d_h), edge_index, batch=batch, with_deg=True
    )
    h, rdeg = _combine1(
        s1p.reshape(NC, p, 8 * d_h), degp.reshape(NC, p, 8 * d_h), z1
    )
    (s2p,) = _sc_segment_sum(
        h.reshape(n_pad, d_h), edge_index, batch=batch, with_deg=False
    )
    Wcat = jnp.concatenate([_blockdiag8(W2_l), _blockdiag8(W2_r)], axis=0)
    return _combine2(
        s2p.reshape(NC, p, 8 * d_h), rdeg, h, Wcat, b2, n, W2_l.shape[1]
    )


# speedup vs baseline: 1.9868x; 1.0081x over previous
import functools

import jax
import jax.numpy as jnp
from jax import lax
from jax.experimental import pallas as pl
from jax.experimental.pallas import tpu as pltpu
from jax.experimental.pallas import tpu_sc as plsc

NC = 2
NS = 16
NW = NC * NS


def _proj1(x, W1_l, W1_r, b1, n_pad):
    n, d_in = x.shape
    d_h = W1_l.shape[1]
    p = n_pad // 8

    def body(x_ref, wl_ref, wr_ref, b_ref, y_ref, z_ref):
        xp = jnp.concatenate(
            [x_ref[...], jnp.zeros((n_pad - n, d_in), jnp.float32)], axis=0
        ).reshape(p, 8, d_in)
        bb = b_ref[...]
        ys = []
        zs = []
        for s in range(8):
            xs = xp[:, s, :]
            ys.append(jnp.dot(xs, wl_ref[...], preferred_element_type=jnp.float32))
            zs.append(
                jnp.dot(xs, wr_ref[...], preferred_element_type=jnp.float32) + bb
            )
        y_ref[...] = jnp.concatenate(ys, axis=1)
        z_ref[...] = jnp.concatenate(zs, axis=1)

    return pl.pallas_call(
        body,
        out_shape=[
            jax.ShapeDtypeStruct((p, 8 * d_h), jnp.float32),
            jax.ShapeDtypeStruct((p, 8 * d_h), jnp.float32),
        ],
    )(x, W1_l, W1_r, b1.reshape(1, d_h))


def _combine1(s1p, degp, z1):
    p, dd = z1.shape

    def body(s_ref, d_ref, z_ref, h_ref, r_ref):
        s = s_ref[0] + s_ref[1]
        r = 1.0 / jnp.maximum(d_ref[0] + d_ref[1], 1.0)
        h_ref[...] = jnp.maximum(s * r + z_ref[...], 0.0)
        r_ref[...] = r

    return pl.pallas_call(
        body,
        out_shape=[
            jax.ShapeDtypeStruct((p, dd), jnp.float32),
            jax.ShapeDtypeStruct((p, dd), jnp.float32),
        ],
    )(s1p, degp, z1)


def _blockdiag8(W):
    d_h, d_out = W.shape
    eye = jnp.eye(8, dtype=W.dtype)
    return (eye[:, None, :, None] * W[None, :, None, :]).reshape(
        8 * d_h, 8 * d_out
    )


def _combine2(s2p, rdeg, h, Wcat, b2, n, d_out):
    p = rdeg.shape[0]
    n_pad = p * 8

    def body(s_ref, r_ref, h_ref, w_ref, b_ref, o_ref):
        m = (s_ref[0] + s_ref[1]) * r_ref[...]
        mh = jnp.concatenate([m, h_ref[...]], axis=1)
        res = jnp.dot(mh, w_ref[...], preferred_element_type=jnp.float32)
        out = jnp.stack(
            [res[:, s * d_out:(s + 1) * d_out] for s in range(8)], axis=1
        ).reshape(n_pad, d_out)
        o_ref[...] = out[0:n] + b_ref[...]

    return pl.pallas_call(
        body,
        out_shape=jax.ShapeDtypeStruct((n, d_out), jnp.float32),
    )(s2p, rdeg, h, Wcat, b2.reshape(1, d_out))


@functools.partial(jax.jit, static_argnames=("batch", "with_deg"))
def _sc_segment_sum(y, edge, *, batch, with_deg):
    n, d = y.shape
    e = edge.shape[1]
    per_tile = e // NW
    num_chunks = per_tile // batch
    rows_per_tile = n // NS
    assert num_chunks % 2 == 0 and batch % 8 == 0 and per_tile % batch == 0

    mesh = plsc.VectorSubcoreMesh(
        core_axis_name="c", subcore_axis_name="s", num_cores=NC, num_subcores=NS
    )
    out_type = [jax.ShapeDtypeStruct((NC, n, d), jnp.float32)]
    scratch = [
        pltpu.VMEM((per_tile,), jnp.int32),
        pltpu.VMEM((batch,), jnp.int32),
        pltpu.VMEM((batch,), jnp.int32),
        pltpu.VMEM((batch, d), jnp.float32),
        pltpu.VMEM((batch, d), jnp.float32),
        pltpu.VMEM((rows_per_tile, d), jnp.float32),
        pltpu.VMEM_SHARED((n, d), jnp.float32),
        pltpu.VMEM_SHARED((n, d), jnp.float32),
        pltpu.SemaphoreType.DMA,
        pltpu.SemaphoreType.DMA,
        pltpu.SemaphoreType.DMA,
        pltpu.SemaphoreType.DMA,
    ]
    if with_deg:
        out_type.append(jax.ShapeDtypeStruct((NC, n, d), jnp.float32))
        scratch.append(pltpu.VMEM((batch, d), jnp.float32))
        scratch.append(pltpu.VMEM_SHARED((n, d), jnp.float32))
        scratch.append(pltpu.SemaphoreType.DMA)
        scratch.append(pltpu.SemaphoreType.DMA)

    @functools.partial(
        pl.kernel, out_type=out_type, mesh=mesh, scratch_types=scratch,
        compiler_params=pltpu.CompilerParams(use_tc_tiling_on_sc=False),
    )
    def k(y_hbm, edge_hbm, *refs):
        if with_deg:
            (s_out, deg_out, idxs_v, idxd_a, idxd_b, rows_a, rows_b, zbuf_v,
             acc_s, y_sp, sem_a, sem_b, ssem_a, ssem_b, ones_v, acc_d,
             osem_a, osem_b) = refs
        else:
            (s_out, idxs_v, idxd_a, idxd_b, rows_a, rows_b, zbuf_v,
             acc_s, y_sp, sem_a, sem_b, ssem_a, ssem_b) = refs

        cid = lax.axis_index("c")
        sid = lax.axis_index("s")
        wid = cid * NS + sid
        ebase = wid * per_tile
        base = sid * rows_per_tile

        zv = jnp.zeros((d,), jnp.float32)

        @pl.loop(0, rows_per_tile)
        def _(i):
            zbuf_v[i, :] = zv

        if with_deg:
            ov = jnp.ones((d,), jnp.float32)

            @pl.loop(0, batch)
            def _(i):
                ones_v[i, :] = ov

        pltpu.sync_copy(zbuf_v, acc_s.at[pl.ds(base, rows_per_tile)])
        if with_deg:
            pltpu.sync_copy(zbuf_v, acc_d.at[pl.ds(base, rows_per_tile)])
        pltpu.sync_copy(edge_hbm.at[0, pl.ds(ebase, per_tile)], idxs_v)
        pltpu.sync_copy(edge_hbm.at[1, pl.ds(ebase, batch)], idxd_a)
        nsl = pl.ds(base, rows_per_tile)
        pltpu.sync_copy(y_hbm.at[nsl], y_sp.at[nsl])
        plsc.subcore_barrier()

        pltpu.async_copy(y_sp.at[idxs_v.at[pl.ds(0, batch)]], rows_a, sem_a)

        @pl.loop(0, num_chunks, step=2)
        def _(kk):
            @pl.when(kk > 0)
            def _():
                pltpu.make_async_copy(rows_b, acc_s.at[idxd_b], ssem_b).wait()
                if with_deg:
                    pltpu.make_async_copy(
                        ones_v, acc_d.at[idxd_b], osem_b
                    ).wait()

            pltpu.sync_copy(
                edge_hbm.at[1, pl.ds(ebase + (kk + 1) * batch, batch)], idxd_b
            )
            pltpu.async_copy(
                y_sp.at[idxs_v.at[pl.ds((kk + 1) * batch, batch)]],
                rows_b, sem_b,
            )
            pltpu.make_async_copy(
                y_sp.at[idxs_v.at[pl.ds(kk * batch, batch)]], rows_a, sem_a
            ).wait()
            pltpu.async_copy(rows_a, acc_s.at[idxd_a], ssem_a, add=True)
            if with_deg:
                pltpu.async_copy(ones_v, acc_d.at[idxd_a], osem_a, add=True)

            pltpu.make_async_copy(
                y_sp.at[idxs_v.at[pl.ds((kk + 1) * batch, batch)]],
                rows_b, sem_b,
            ).wait()
            pltpu.async_copy(rows_b, acc_s.at[idxd_b], ssem_b, add=True)
            if with_deg:
                pltpu.async_copy(ones_v, acc_d.at[idxd_b], osem_b, add=True)

            pltpu.make_async_copy(rows_a, acc_s.at[idxd_a], ssem_a).wait()
            if with_deg:
                pltpu.make_async_copy(ones_v, acc_d.at[idxd_a], osem_a).wait()

            @pl.when(kk + 2 < num_chunks)
            def _():
                pltpu.sync_copy(
                    edge_hbm.at[1, pl.ds(ebase + (kk + 2) * batch, batch)],
                    idxd_a,
                )
                pltpu.async_copy(
                    y_sp.at[idxs_v.at[pl.ds((kk + 2) * batch, batch)]],
                    rows_a, sem_a,
                )

        pltpu.make_async_copy(rows_b, acc_s.at[idxd_b], ssem_b).wait()
        if with_deg:
            pltpu.make_async_copy(ones_v, acc_d.at[idxd_b], osem_b).wait()

        plsc.subcore_barrier()
        sl = pl.ds(base, rows_per_tile)
        pltpu.sync_copy(acc_s.at[sl], s_out.at[cid, sl])
        if with_deg:
            pltpu.sync_copy(acc_d.at[sl], deg_out.at[cid, sl])

    return k(y, edge)


def kernel(x, edge_index, W1_l, b1, W1_r, W2_l, b2, W2_r):
    n = x.shape[0]
    e = edge_index.shape[1]
    per_tile = e // NW
    assert per_tile * NW == e
    batch = max(
        b for b in range(8, 1256, 8)
        if per_tile % b == 0 and (per_tile // b) % 2 == 0
    )
    n_pad = -(-n // (NS * 8)) * (NS * 8)

    d_h = W1_l.shape[1]
    p = n_pad // 8

    y1, z1 = _proj1(x, W1_l, W1_r, b1, n_pad)
    s1p, degp = _sc_segment_sum(
        y1.reshape(n_pad, d_h), edge_index, batch=batch, with_deg=True
    )
    h, rdeg = _combine1(
        s1p.reshape(NC, p, 8 * d_h), degp.reshape(NC, p, 8 * d_h), z1
    )
    (s2p,) = _sc_segment_sum(
        h.reshape(n_pad, d_h), edge_index, batch=batch, with_deg=False
    )
    Wcat = jnp.concatenate([_blockdiag8(W2_l), _blockdiag8(W2_r)], axis=0)
    return _combine2(
        s2p.reshape(NC, p, 8 * d_h), rdeg, h, Wcat, b2, n, W2_l.shape[1]
    )
